# baseline (speedup 1.0000x reference)
import functools

import jax
import jax.numpy as jnp
from jax import lax
from jax.experimental import pallas as pl
from jax.experimental.pallas import tpu as pltpu, tpu_sc as plsc

N = 10000
D = 128
E = 320000

NC = 2
NS = 16
NW = NC * NS

EPT = E // NW
K = 128
FULL = EPT // K
TAIL = EPT - FULL * K
N_PAD = 10240
RPT = N_PAD // NS


def _sc_deg_body(rows, ones_hbm, zeros_hbm, deg_out, ring_v, ones_v, tail_v,
                 deg_sh, ss, i0, i1, i2, i3, i4, i5, i6, i7):
    c = lax.axis_index("c")
    s = lax.axis_index("s")
    base = (c * NS + s) * EPT
    isems = (i0, i1, i2, i3, i4, i5, i6, i7)
    pltpu.sync_copy(zeros_hbm, deg_sh.at[pl.ds(s * RPT, RPT)])
    pltpu.sync_copy(ones_hbm, ones_v)
    for t in range(8):
        pltpu.async_copy(rows.at[pl.ds(base + t * K, K)],
                         ring_v.at[t], isems[t])
    plsc.subcore_barrier()

    def _group(g, slot0):
        for k in range(4):
            pltpu.make_async_copy(
                rows.at[pl.ds(base + (4 * g + k) * K, K)],
                ring_v.at[slot0 + k], isems[slot0 + k]).wait()
        for k in range(4):
            pltpu.async_copy(ones_v, deg_sh.at[ring_v.at[slot0 + k]], ss,
                             add=True)
        for k in range(4):
            pltpu.make_async_copy(ones_v, deg_sh.at[ring_v.at[slot0 + k]],
                                  ss).wait()
        for k in range(4):
            j2 = (g + 2) * 4 + k

            @pl.when(j2 < FULL)
            def _():
                pltpu.async_copy(rows.at[pl.ds(base + j2 * K, K)],
                                 ring_v.at[slot0 + k], isems[slot0 + k])

    def body(g2, carry):
        _group(2 * g2, 0)
        _group(2 * g2 + 1, 4)
        return carry

    lax.fori_loop(0, FULL // 8, body, 0)
    for j in range(FULL - 6, FULL):
        slot = j - (FULL - 6)
        pltpu.make_async_copy(rows.at[pl.ds(base + j * K, K)],
                              ring_v.at[slot], isems[slot]).wait()
        pltpu.async_copy(ones_v, deg_sh.at[ring_v.at[slot]], ss, add=True)
    for j in range(FULL - 6, FULL):
        slot = j - (FULL - 6)
        pltpu.make_async_copy(ones_v, deg_sh.at[ring_v.at[slot]], ss).wait()
    pltpu.sync_copy(rows.at[pl.ds(base + FULL * K, TAIL)], tail_v)
    pltpu.sync_copy(ones_v.at[pl.ds(0, TAIL)], deg_sh.at[tail_v], add=True)
    plsc.subcore_barrier()

    @pl.when(s == 0)
    def _():
        pltpu.sync_copy(deg_sh, deg_out.at[c])


def _sc_agg_body(y_hbm, rows, col_hbm, zeros_hbm, acc_out,
                 rring_v, cring_v, tailr_v, tailc_v, vals0_v, vals1_v, acc_sh,
                 gs0, gs1, rs0, rs1, rs2, rs3, cs0, cs1, cs2, cs3):
    c = lax.axis_index("c")
    s = lax.axis_index("s")
    rbase = (c * NS + s) * EPT
    vbufs = (vals0_v, vals1_v)
    gsems = (gs0, gs1)
    rsems = (rs0, rs1, rs2, rs3)
    csems = (cs0, cs1, cs2, cs3)
    pltpu.sync_copy(zeros_hbm, acc_sh.at[pl.ds(s * RPT, RPT), :])
    for t in range(4):
        pltpu.async_copy(rows.at[pl.ds(rbase + t * K, K)],
                         rring_v.at[t], rsems[t])
        pltpu.async_copy(col_hbm.at[pl.ds(rbase + t * K, K)],
                         cring_v.at[t], csems[t])
    for t in range(2):
        pltpu.make_async_copy(rows.at[pl.ds(rbase + t * K, K)],
                              rring_v.at[t], rsems[t]).wait()
        pltpu.async_copy(y_hbm.at[rring_v.at[t]], vbufs[t], gsems[t])
    plsc.subcore_barrier()

    def body(j4, carry):
        for k in range(4):
            j = 4 * j4 + k
            buf, gsm = vbufs[k % 2], gsems[k % 2]
            pltpu.make_async_copy(y_hbm.at[rring_v.at[k]], buf, gsm).wait()

            @pl.when(j + 4 < FULL)
            def _():
                pltpu.async_copy(rows.at[pl.ds(rbase + (j + 4) * K, K)],
                                 rring_v.at[k], rsems[k])

            pltpu.make_async_copy(col_hbm.at[pl.ds(rbase + j * K, K)],
                                  cring_v.at[k], csems[k]).wait()
            pltpu.sync_copy(buf, acc_sh.at[cring_v.at[k]], add=True)

            @pl.when(j + 4 < FULL)
            def _():
                pltpu.async_copy(col_hbm.at[pl.ds(rbase + (j + 4) * K, K)],
                                 cring_v.at[k], csems[k])

            @pl.when(j + 2 < FULL)
            def _():
                pltpu.make_async_copy(
                    rows.at[pl.ds(rbase + (j + 2) * K, K)],
                    rring_v.at[(k + 2) % 4], rsems[(k + 2) % 4]).wait()
                pltpu.async_copy(y_hbm.at[rring_v.at[(k + 2) % 4]], buf, gsm)

        return carry

    lax.fori_loop(0, FULL // 4, body, 0)
    for j in (FULL - 2, FULL - 1):
        k = j % 4
        buf, gsm = vbufs[j % 2], gsems[j % 2]
        pltpu.make_async_copy(y_hbm.at[rring_v.at[k]], buf, gsm).wait()
        pltpu.make_async_copy(col_hbm.at[pl.ds(rbase + j * K, K)],
                              cring_v.at[k], csems[k]).wait()
        pltpu.sync_copy(buf, acc_sh.at[cring_v.at[k]], add=True)
    pltpu.sync_copy(rows.at[pl.ds(rbase + FULL * K, TAIL)], tailr_v)
    pltpu.sync_copy(col_hbm.at[pl.ds(rbase + FULL * K, TAIL)], tailc_v)
    pltpu.sync_copy(y_hbm.at[tailr_v], vals0_v.at[pl.ds(0, TAIL), :])
    pltpu.sync_copy(vals0_v.at[pl.ds(0, TAIL), :], acc_sh.at[tailc_v],
                    add=True)
    plsc.subcore_barrier()
    pltpu.sync_copy(acc_sh.at[pl.ds(s * RPT, RPT), :],
                    acc_out.at[c, pl.ds(s * RPT, RPT), :])


@functools.cache
def _build_sc_kernels():
    mesh = plsc.VectorSubcoreMesh(core_axis_name="c", subcore_axis_name="s",
                                  num_cores=NC, num_subcores=NS)
    sc_deg = pl.kernel(
        _sc_deg_body,
        mesh=mesh,
        out_type=jax.ShapeDtypeStruct((NC, N_PAD), jnp.float32),
        scratch_types=[
            pltpu.VMEM((8, K), jnp.int32),
            pltpu.VMEM((K,), jnp.float32),
            pltpu.VMEM((TAIL,), jnp.int32),
            pltpu.VMEM_SHARED((N_PAD,), jnp.float32),
        ] + [pltpu.SemaphoreType.DMA] * 9,
    )
    sc_agg = pl.kernel(
        _sc_agg_body,
        mesh=mesh,
        out_type=jax.ShapeDtypeStruct((NC, N_PAD, D), jnp.float32),
        scratch_types=[
            pltpu.VMEM((4, K), jnp.int32),
            pltpu.VMEM((4, K), jnp.int32),
            pltpu.VMEM((TAIL,), jnp.int32),
            pltpu.VMEM((TAIL,), jnp.int32),
            pltpu.VMEM((K, D), jnp.float32),
            pltpu.VMEM((K, D), jnp.float32),
            pltpu.VMEM_SHARED((N_PAD, D), jnp.float32),
        ] + [pltpu.SemaphoreType.DMA] * 10,
    )
    return sc_deg, sc_agg


def _tc_prep_body(degp_ref, x_ref, y_ref):
    dp = degp_ref[...]
    dsum = (dp[0:1, :] + dp[1:2, :]).reshape(N_PAD, 1)
    deg = (1.0 + dsum)[:N, :]
    y_ref[...] = lax.rsqrt(deg) * x_ref[...]


def _tc_post_body(acc_ref, degp_ref, x_ref, w_ref, b_ref, g_ref, bt_ref, o_ref):
    a = acc_ref[0, :, :] + acc_ref[1, :, :]
    a = a[:N, :]
    dp = degp_ref[...]
    dsum = (dp[0:1, :] + dp[1:2, :]).reshape(N_PAD, 1)
    deg = (1.0 + dsum)[:N, :]
    dis = lax.rsqrt(deg)
    x = x_ref[...]
    z = dis * a + (dis * dis) * x
    agg = lax.dot_general(z, w_ref[...], (((1,), (1,)), ((), ())),
                          preferred_element_type=jnp.float32)
    agg = agg + b_ref[...]
    mean = jnp.mean(agg, axis=0, keepdims=True)
    var = jnp.mean((agg - mean) ** 2, axis=0, keepdims=True)
    h = (agg - mean) * lax.rsqrt(var + 1e-5) * g_ref[...] + bt_ref[...]
    h = jnp.where(h >= 0, h, 0.1 * h)
    out = h + x
    o_ref[...] = jnp.where(out >= 0, out, 0.1 * out)


_tc_prep = pl.pallas_call(
    _tc_prep_body,
    out_shape=jax.ShapeDtypeStruct((N, D), jnp.float32),
)

_tc_post = pl.pallas_call(
    _tc_post_body,
    out_shape=jax.ShapeDtypeStruct((N, D), jnp.float32),
)


def kernel(x, W, b, bn_gamma, bn_beta, edge_index):
    ei2 = edge_index.astype(jnp.int32)
    row1d = ei2[0]
    col1d = ei2[1]

    ones_k = jnp.ones((K,), jnp.float32)
    zeros_1d = jnp.zeros((RPT,), jnp.float32)
    zeros_2d = jnp.zeros((RPT, D), jnp.float32)

    sc_deg, sc_agg = _build_sc_kernels()
    deg_parts = sc_deg(row1d, ones_k, zeros_1d)
    y = _tc_prep(deg_parts, x)
    acc_parts = sc_agg(y, row1d, col1d, zeros_2d)
    return _tc_post(acc_parts, deg_parts, x,
                    W, b.reshape(1, D), bn_gamma.reshape(1, D),
                    bn_beta.reshape(1, D))


# speedup vs baseline: 1.0727x; 1.0727x over previous
import functools

import jax
import jax.numpy as jnp
from jax import lax
from jax.experimental import pallas as pl
from jax.experimental.pallas import tpu as pltpu, tpu_sc as plsc

N = 10000
D = 128
E = 320000

NC = 2
NS = 16
NW = NC * NS

EPT = E // NW
K = 128
FULL = EPT // K
TAIL = EPT - FULL * K
N_PAD = 10240
RPT = N_PAD // NS


def _sc_deg_body(ei_hbm, ones_hbm, zeros_hbm, deg_out, ring_v, ones_v, tail_v,
                 deg_sh, ss, i0, i1, i2, i3, i4, i5, i6, i7):
    c = lax.axis_index("c")
    s = lax.axis_index("s")
    base = (c * NS + s) * EPT
    rows = ei_hbm
    isems = (i0, i1, i2, i3, i4, i5, i6, i7)
    pltpu.sync_copy(zeros_hbm, deg_sh.at[pl.ds(s * RPT, RPT)])
    pltpu.sync_copy(ones_hbm, ones_v)
    for t in range(8):
        pltpu.async_copy(rows.at[pl.ds(base + t * K, K)],
                         ring_v.at[t], isems[t])
    plsc.subcore_barrier()

    def _group(g, slot0):
        for k in range(4):
            pltpu.make_async_copy(
                rows.at[pl.ds(base + (4 * g + k) * K, K)],
                ring_v.at[slot0 + k], isems[slot0 + k]).wait()
        for k in range(4):
            pltpu.async_copy(ones_v, deg_sh.at[ring_v.at[slot0 + k]], ss,
                             add=True)
        for k in range(4):
            pltpu.make_async_copy(ones_v, deg_sh.at[ring_v.at[slot0 + k]],
                                  ss).wait()
        for k in range(4):
            j2 = (g + 2) * 4 + k

            @pl.when(j2 < FULL)
            def _():
                pltpu.async_copy(rows.at[pl.ds(base + j2 * K, K)],
                                 ring_v.at[slot0 + k], isems[slot0 + k])

    def body(g2, carry):
        _group(2 * g2, 0)
        _group(2 * g2 + 1, 4)
        return carry

    lax.fori_loop(0, FULL // 8, body, 0)
    for j in range(FULL - 6, FULL):
        slot = j - (FULL - 6)
        pltpu.make_async_copy(rows.at[pl.ds(base + j * K, K)],
                              ring_v.at[slot], isems[slot]).wait()
        pltpu.async_copy(ones_v, deg_sh.at[ring_v.at[slot]], ss, add=True)
    for j in range(FULL - 6, FULL):
        slot = j - (FULL - 6)
        pltpu.make_async_copy(ones_v, deg_sh.at[ring_v.at[slot]], ss).wait()
    pltpu.sync_copy(rows.at[pl.ds(base + FULL * K, TAIL)], tail_v)
    pltpu.sync_copy(ones_v.at[pl.ds(0, TAIL)], deg_sh.at[tail_v], add=True)
    plsc.subcore_barrier()

    @pl.when(s == 0)
    def _():
        pltpu.sync_copy(deg_sh, deg_out.at[c])


def _sc_agg_body(y_hbm, ei_hbm, zeros_hbm, acc_out,
                 rring_v, cring_v, tailr_v, tailc_v, vals0_v, vals1_v, acc_sh,
                 gs0, gs1, rs0, rs1, rs2, rs3, cs0, cs1, cs2, cs3):
    c = lax.axis_index("c")
    s = lax.axis_index("s")
    rbase = (c * NS + s) * EPT
    cbase = E + rbase
    rows = ei_hbm
    vbufs = (vals0_v, vals1_v)
    gsems = (gs0, gs1)
    rsems = (rs0, rs1, rs2, rs3)
    csems = (cs0, cs1, cs2, cs3)
    pltpu.sync_copy(zeros_hbm, acc_sh.at[pl.ds(s * RPT, RPT), :])
    for t in range(4):
        pltpu.async_copy(rows.at[pl.ds(rbase + t * K, K)],
                         rring_v.at[t], rsems[t])
        pltpu.async_copy(ei_hbm.at[pl.ds(cbase + t * K, K)],
                         cring_v.at[t], csems[t])
    for t in range(2):
        pltpu.make_async_copy(rows.at[pl.ds(rbase + t * K, K)],
                              rring_v.at[t], rsems[t]).wait()
        pltpu.async_copy(y_hbm.at[rring_v.at[t]], vbufs[t], gsems[t])
    plsc.subcore_barrier()

    def body(j4, carry):
        for k in range(4):
            j = 4 * j4 + k
            buf, gsm = vbufs[k % 2], gsems[k % 2]
            pltpu.make_async_copy(y_hbm.at[rring_v.at[k]], buf, gsm).wait()

            @pl.when(j + 4 < FULL)
            def _():
                pltpu.async_copy(rows.at[pl.ds(rbase + (j + 4) * K, K)],
                                 rring_v.at[k], rsems[k])

            pltpu.make_async_copy(ei_hbm.at[pl.ds(cbase + j * K, K)],
                                  cring_v.at[k], csems[k]).wait()
            pltpu.sync_copy(buf, acc_sh.at[cring_v.at[k]], add=True)

            @pl.when(j + 4 < FULL)
            def _():
                pltpu.async_copy(ei_hbm.at[pl.ds(cbase + (j + 4) * K, K)],
                                 cring_v.at[k], csems[k])

            @pl.when(j + 2 < FULL)
            def _():
                pltpu.make_async_copy(
                    rows.at[pl.ds(rbase + (j + 2) * K, K)],
                    rring_v.at[(k + 2) % 4], rsems[(k + 2) % 4]).wait()
                pltpu.async_copy(y_hbm.at[rring_v.at[(k + 2) % 4]], buf, gsm)

        return carry

    lax.fori_loop(0, FULL // 4, body, 0)
    for j in (FULL - 2, FULL - 1):
        k = j % 4
        buf, gsm = vbufs[j % 2], gsems[j % 2]
        pltpu.make_async_copy(y_hbm.at[rring_v.at[k]], buf, gsm).wait()
        pltpu.make_async_copy(ei_hbm.at[pl.ds(cbase + j * K, K)],
                              cring_v.at[k], csems[k]).wait()
        pltpu.sync_copy(buf, acc_sh.at[cring_v.at[k]], add=True)
    pltpu.sync_copy(rows.at[pl.ds(rbase + FULL * K, TAIL)], tailr_v)
    pltpu.sync_copy(ei_hbm.at[pl.ds(cbase + FULL * K, TAIL)], tailc_v)
    pltpu.sync_copy(y_hbm.at[tailr_v], vals0_v.at[pl.ds(0, TAIL), :])
    pltpu.sync_copy(vals0_v.at[pl.ds(0, TAIL), :], acc_sh.at[tailc_v],
                    add=True)
    plsc.subcore_barrier()
    pltpu.sync_copy(acc_sh.at[pl.ds(s * RPT, RPT), :],
                    acc_out.at[c, pl.ds(s * RPT, RPT), :])


@functools.cache
def _build_sc_kernels():
    mesh = plsc.VectorSubcoreMesh(core_axis_name="c", subcore_axis_name="s",
                                  num_cores=NC, num_subcores=NS)
    sc_deg = pl.kernel(
        _sc_deg_body,
        mesh=mesh,
        out_type=jax.ShapeDtypeStruct((NC, N_PAD), jnp.float32),
        scratch_types=[
            pltpu.VMEM((8, K), jnp.int32),
            pltpu.VMEM((K,), jnp.float32),
            pltpu.VMEM((TAIL,), jnp.int32),
            pltpu.VMEM_SHARED((N_PAD,), jnp.float32),
        ] + [pltpu.SemaphoreType.DMA] * 9,
    )
    sc_agg = pl.kernel(
        _sc_agg_body,
        mesh=mesh,
        out_type=jax.ShapeDtypeStruct((NC, N_PAD, D), jnp.float32),
        scratch_types=[
            pltpu.VMEM((4, K), jnp.int32),
            pltpu.VMEM((4, K), jnp.int32),
            pltpu.VMEM((TAIL,), jnp.int32),
            pltpu.VMEM((TAIL,), jnp.int32),
            pltpu.VMEM((K, D), jnp.float32),
            pltpu.VMEM((K, D), jnp.float32),
            pltpu.VMEM_SHARED((N_PAD, D), jnp.float32),
        ] + [pltpu.SemaphoreType.DMA] * 10,
    )
    return sc_deg, sc_agg


def _tc_prep_body(degp_ref, x_ref, y_ref):
    dp = degp_ref[...]
    dsum = (dp[0:1, :] + dp[1:2, :]).reshape(N_PAD, 1)
    deg = (1.0 + dsum)[:N, :]
    y_ref[...] = lax.rsqrt(deg) * x_ref[...]


def _tc_post_body(acc_ref, degp_ref, x_ref, w_ref, b_ref, g_ref, bt_ref, o_ref):
    a = acc_ref[0, :, :] + acc_ref[1, :, :]
    a = a[:N, :]
    dp = degp_ref[...]
    dsum = (dp[0:1, :] + dp[1:2, :]).reshape(N_PAD, 1)
    deg = (1.0 + dsum)[:N, :]
    dis = lax.rsqrt(deg)
    x = x_ref[...]
    z = dis * a + (dis * dis) * x
    agg = lax.dot_general(z, w_ref[...], (((1,), (1,)), ((), ())),
                          preferred_element_type=jnp.float32)
    agg = agg + b_ref[...]
    mean = jnp.mean(agg, axis=0, keepdims=True)
    var = jnp.mean((agg - mean) ** 2, axis=0, keepdims=True)
    h = (agg - mean) * lax.rsqrt(var + 1e-5) * g_ref[...] + bt_ref[...]
    h = jnp.where(h >= 0, h, 0.1 * h)
    out = h + x
    o_ref[...] = jnp.where(out >= 0, out, 0.1 * out)


_tc_prep = pl.pallas_call(
    _tc_prep_body,
    out_shape=jax.ShapeDtypeStruct((N, D), jnp.float32),
)

_tc_post = pl.pallas_call(
    _tc_post_body,
    out_shape=jax.ShapeDtypeStruct((N, D), jnp.float32),
)


def kernel(x, W, b, bn_gamma, bn_beta, edge_index):
    ei1d = edge_index.astype(jnp.int32).reshape(2 * E)

    ones_k = jnp.ones((K,), jnp.float32)
    zeros_1d = jnp.zeros((RPT,), jnp.float32)
    zeros_2d = jnp.zeros((RPT, D), jnp.float32)

    sc_deg, sc_agg = _build_sc_kernels()
    deg_parts = sc_deg(ei1d, ones_k, zeros_1d)
    y = _tc_prep(deg_parts, x)
    acc_parts = sc_agg(y, ei1d, zeros_2d)
    return _tc_post(acc_parts, deg_parts, x,
                    W, b.reshape(1, D), bn_gamma.reshape(1, D),
                    bn_beta.reshape(1, D))
